# baseline (speedup 1.0000x reference)
import functools

import jax
import jax.numpy as jnp
from jax import lax
from jax.experimental import pallas as pl
from jax.experimental.pallas import tpu as pltpu
from jax.experimental.pallas import tpu_sc as plsc

N_NODES = 10000
D = 128
DO = 64
NC, NS, LANES = 2, 16, 16
NW = NC * NS
EB = 128
C = 16
R = 10240
RS = R // NS
ROWB = 400

_mesh = plsc.VectorSubcoreMesh(core_axis_name="c", subcore_axis_name="s")

_CHUNKS = [(q * EB, EB) for q in range(RS // EB)]
if RS % EB:
    _CHUNKS.append((RS - RS % EB, RS % EB))
_ZROWS = 64
_DCHUNKS = [(q * _ZROWS, _ZROWS) for q in range(RS // _ZROWS)]


@functools.cache
def _make_deg_kernel(nb):
    @functools.partial(
        pl.kernel,
        mesh=_mesh,
        out_type=jax.ShapeDtypeStruct((NC, R, LANES), jnp.float32),
        scratch_types=[
            pltpu.VMEM((nb, EB), jnp.int32),
            pltpu.VMEM((EB, LANES), jnp.float32),
            pltpu.VMEM((_ZROWS, LANES), jnp.float32),
            pltpu.VMEM_SHARED((R, LANES), jnp.float32),
        ],
    )
    def deg_kernel(dst_hbm, out_hbm, idx_v, ones_v, zeros_v, deg_sh):
        c = lax.axis_index("c")
        s = lax.axis_index("s")
        w = c * NS + s

        def _fill(i, carry):
            ones_v[i, :] = jnp.ones((LANES,), jnp.float32)

            @pl.when(i < _ZROWS)
            def _():
                zeros_v[i, :] = jnp.zeros((LANES,), jnp.float32)

            return carry

        lax.fori_loop(0, EB, _fill, 0)

        base = s * RS
        for off, ln in _DCHUNKS:
            pltpu.sync_copy(zeros_v.at[pl.ds(0, ln)],
                            deg_sh.at[pl.ds(base + off, ln)])
        pltpu.sync_copy(dst_hbm.at[w], idx_v)
        plsc.subcore_barrier()

        def _acc(j, carry):
            pltpu.sync_copy(ones_v, deg_sh.at[idx_v.at[j]], add=True)
            return carry

        lax.fori_loop(0, nb, _acc, 0)
        plsc.subcore_barrier()
        pltpu.sync_copy(deg_sh.at[pl.ds(base, RS)],
                        out_hbm.at[c, pl.ds(base, RS)])

    return deg_kernel


@functools.cache
def _make_agg_kernel(nb):
    nch = nb // C

    @functools.partial(
        pl.kernel,
        mesh=_mesh,
        out_type=jax.ShapeDtypeStruct((NC, R, D), jnp.float32),
        scratch_types=[
            pltpu.VMEM((2, C, EB), jnp.int32),
            pltpu.VMEM((2, C, EB), jnp.int32),
            pltpu.VMEM((EB, D), jnp.float32),
            pltpu.VMEM((EB, D), jnp.float32),
            pltpu.VMEM_SHARED((R, D), jnp.float32),
            pltpu.SemaphoreType.DMA,
            pltpu.SemaphoreType.DMA,
            pltpu.SemaphoreType.DMA,
            pltpu.SemaphoreType.DMA,
            pltpu.SemaphoreType.DMA,
        ],
    )
    def agg_kernel(src_hbm, dst_hbm, h2_hbm, out_hbm,
                   srcc, dstc, bufa, bufb, agg_sh, sema, semb, ssa, ssb, semi):
        c = lax.axis_index("c")
        s = lax.axis_index("s")
        w = c * NS + s

        def _zero(i, carry):
            for k in range(D // LANES):
                bufa[i, pl.ds(k * LANES, LANES)] = jnp.zeros((LANES,),
                                                             jnp.float32)
            return carry

        lax.fori_loop(0, EB, _zero, 0)
        base = s * RS
        for off, ln in _CHUNKS:
            pltpu.sync_copy(bufa.at[pl.ds(0, ln)],
                            agg_sh.at[pl.ds(base + off, ln)])
        pltpu.sync_copy(src_hbm.at[w, pl.ds(0, C)], srcc.at[0])
        pltpu.sync_copy(dst_hbm.at[w, pl.ds(0, C)], dstc.at[0])
        plsc.subcore_barrier()

        def _gref(j):
            return srcc.at[lax.rem(lax.div(j, C), 2), lax.rem(j, C)]

        def _sref(j):
            return dstc.at[lax.rem(lax.div(j, C), 2), lax.rem(j, C)]

        pltpu.async_copy(h2_hbm.at[_gref(0)], bufa, sema)
        pltpu.async_copy(h2_hbm.at[_gref(1)], bufb, semb)

        def _pair(t, carry):
            j = t * 2
            g = lax.div(j, C)

            @pl.when(jnp.logical_and(lax.rem(j, C) == 0, g + 1 < nch))
            def _():
                pltpu.async_copy(src_hbm.at[w, pl.ds((g + 1) * C, C)],
                                 srcc.at[lax.rem(g + 1, 2)], semi)
                pltpu.async_copy(dst_hbm.at[w, pl.ds((g + 1) * C, C)],
                                 dstc.at[lax.rem(g + 1, 2)], semi)

            pltpu.make_async_copy(h2_hbm.at[_gref(j)], bufa, sema).wait()
            pltpu.async_copy(bufa, agg_sh.at[_sref(j)], ssa, add=True)
            pltpu.make_async_copy(h2_hbm.at[_gref(j + 1)], bufb, semb).wait()
            pltpu.async_copy(bufb, agg_sh.at[_sref(j + 1)], ssb, add=True)

            @pl.when(jnp.logical_and(lax.rem(j, C) == C - 2, g + 1 < nch))
            def _():
                pltpu.make_async_copy(src_hbm.at[w, pl.ds((g + 1) * C, C)],
                                      srcc.at[lax.rem(g + 1, 2)], semi).wait()
                pltpu.make_async_copy(dst_hbm.at[w, pl.ds((g + 1) * C, C)],
                                      dstc.at[lax.rem(g + 1, 2)], semi).wait()

            pltpu.make_async_copy(bufa, agg_sh.at[_sref(j)], ssa).wait()

            @pl.when(j + 2 < nb)
            def _():
                pltpu.async_copy(h2_hbm.at[_gref(j + 2)], bufa, sema)

            pltpu.make_async_copy(bufb, agg_sh.at[_sref(j + 1)], ssb).wait()

            @pl.when(j + 3 < nb)
            def _():
                pltpu.async_copy(h2_hbm.at[_gref(j + 3)], bufb, semb)

            return carry

        lax.fori_loop(0, nb // 2, _pair, 0)
        plsc.subcore_barrier()
        pltpu.sync_copy(agg_sh.at[pl.ds(base, RS)],
                        out_hbm.at[c, pl.ds(base, RS)])

    return agg_kernel


def _mm_body(x_ref, w_ref, o_ref):
    o_ref[...] = jnp.dot(x_ref[...], w_ref[...],
                         preferred_element_type=jnp.float32)


_mm_call = pl.pallas_call(
    _mm_body,
    grid=(N_NODES // ROWB,),
    in_specs=[
        pl.BlockSpec((ROWB, D), lambda i: (i, 0)),
        pl.BlockSpec((D, D), lambda i: (0, 0)),
    ],
    out_specs=pl.BlockSpec((ROWB, D), lambda i: (i, 0)),
    out_shape=jax.ShapeDtypeStruct((N_NODES, D), jnp.float32),
)


def _h2_body(h_ref, degp_ref, o_ref):
    dg = degp_ref[...]
    deg = dg[0, :, 0:1] + dg[1, :, 0:1] + 1.0
    o_ref[...] = h_ref[...] * lax.rsqrt(deg)


_h2_call = pl.pallas_call(
    _h2_body,
    grid=(N_NODES // ROWB,),
    in_specs=[
        pl.BlockSpec((ROWB, D), lambda i: (i, 0)),
        pl.BlockSpec((NC, ROWB, LANES), lambda i: (0, i, 0)),
    ],
    out_specs=pl.BlockSpec((ROWB, D), lambda i: (i, 0)),
    out_shape=jax.ShapeDtypeStruct((N_NODES, D), jnp.float32),
)


def _mlp_body(aggp_ref, degp_ref, h2_ref, bg_ref, w1_ref, b1_ref, w2_ref,
              b2_ref, o_ref):
    p = aggp_ref[...]
    dg = degp_ref[...]
    deg = dg[0, :, 0:1] + dg[1, :, 0:1] + 1.0
    dinv = lax.rsqrt(deg)
    t = (p[0] + p[1] + h2_ref[...]) * dinv
    a = jnp.maximum(t + bg_ref[...], 0.0)
    m = jnp.maximum(
        jnp.dot(a, w1_ref[...], preferred_element_type=jnp.float32)
        + b1_ref[...], 0.0)
    o = jnp.dot(m, w2_ref[...], preferred_element_type=jnp.float32) + b2_ref[...]
    mx = jnp.max(o, axis=1, keepdims=True)
    lse = mx + jnp.log(jnp.sum(jnp.exp(o - mx), axis=1, keepdims=True))
    o_ref[...] = o - lse


_mlp_call = pl.pallas_call(
    _mlp_body,
    grid=(N_NODES // ROWB,),
    in_specs=[
        pl.BlockSpec((NC, ROWB, D), lambda i: (0, i, 0)),
        pl.BlockSpec((NC, ROWB, LANES), lambda i: (0, i, 0)),
        pl.BlockSpec((ROWB, D), lambda i: (i, 0)),
        pl.BlockSpec((1, D), lambda i: (0, 0)),
        pl.BlockSpec((D, D), lambda i: (0, 0)),
        pl.BlockSpec((1, D), lambda i: (0, 0)),
        pl.BlockSpec((D, DO), lambda i: (0, 0)),
        pl.BlockSpec((1, DO), lambda i: (0, 0)),
    ],
    out_specs=pl.BlockSpec((ROWB, DO), lambda i: (i, 0)),
    out_shape=jax.ShapeDtypeStruct((N_NODES, DO), jnp.float32),
)


def kernel(x, edge_index, W_gcn, b_gcn, W1, b1, W2, b2):
    e = edge_index.shape[1]
    nb = -(-e // (NW * EB))
    nb += (-nb) % C
    padn = NW * nb * EB - e

    ei = edge_index.astype(jnp.int32)
    pidx = jnp.arange(padn, dtype=jnp.int32)
    pad_src = (pidx * 131) % N_NODES
    pad_dst = N_NODES + pidx % (R - N_NODES)
    src = jnp.concatenate([ei[0], pad_src]).reshape(NW, nb, EB)
    dst = jnp.concatenate([ei[1], pad_dst]).reshape(NW, nb, EB)

    h = _mm_call(x, W_gcn)
    degp = _make_deg_kernel(nb)(dst)
    h2 = _h2_call(h, degp)
    aggp = _make_agg_kernel(nb)(src, dst, h2)
    return _mlp_call(aggp, degp, h2, b_gcn.reshape(1, D), W1,
                     b1.reshape(1, D), W2, b2.reshape(1, DO))


# speedup vs baseline: 1.2579x; 1.2579x over previous
import functools

import jax
import jax.numpy as jnp
from jax import lax
from jax.experimental import pallas as pl
from jax.experimental.pallas import tpu as pltpu
from jax.experimental.pallas import tpu_sc as plsc

N_NODES = 10000
D = 128
DO = 64
NC, NS, LANES = 2, 16, 16
NW = NC * NS
EB = 112
C = 8
R = 10240
RA = 10112
RS = R // NS
RAS = RA // NS
ROWB = 400

_mesh = plsc.VectorSubcoreMesh(core_axis_name="c", subcore_axis_name="s")

_CHUNKS = [(q * EB, EB) for q in range(RAS // EB)]
if RAS % EB:
    _CHUNKS.append((RAS - RAS % EB, RAS % EB))
_ZROWS = 64
_DCHUNKS = [(q * _ZROWS, _ZROWS) for q in range(RS // _ZROWS)]


@functools.cache
def _make_deg_kernel(nb, nbp):
    @functools.partial(
        pl.kernel,
        mesh=_mesh,
        out_type=jax.ShapeDtypeStruct((NC, R, LANES), jnp.float32),
        scratch_types=[
            pltpu.VMEM((nbp, EB), jnp.int32),
            pltpu.VMEM((EB, LANES), jnp.float32),
            pltpu.VMEM((_ZROWS, LANES), jnp.float32),
            pltpu.VMEM_SHARED((R, LANES), jnp.float32),
        ],
    )
    def deg_kernel(dst_hbm, out_hbm, idx_v, ones_v, zeros_v, deg_sh):
        c = lax.axis_index("c")
        s = lax.axis_index("s")
        w = c * NS + s

        def _fill(i, carry):
            ones_v[i, :] = jnp.ones((LANES,), jnp.float32)

            @pl.when(i < _ZROWS)
            def _():
                zeros_v[i, :] = jnp.zeros((LANES,), jnp.float32)

            return carry

        lax.fori_loop(0, EB, _fill, 0)

        base = s * RS
        for off, ln in _DCHUNKS:
            pltpu.sync_copy(zeros_v.at[pl.ds(0, ln)],
                            deg_sh.at[pl.ds(base + off, ln)])
        pltpu.sync_copy(dst_hbm.at[w], idx_v)
        plsc.subcore_barrier()

        def _acc(j, carry):
            pltpu.sync_copy(ones_v, deg_sh.at[idx_v.at[j]], add=True)
            return carry

        lax.fori_loop(0, nb, _acc, 0)
        plsc.subcore_barrier()
        pltpu.sync_copy(deg_sh.at[pl.ds(base, RS)],
                        out_hbm.at[c, pl.ds(base, RS)])

    return deg_kernel


@functools.cache
def _make_agg_kernel(nb, nbp):
    nch = nbp // C

    @functools.partial(
        pl.kernel,
        mesh=_mesh,
        out_type=jax.ShapeDtypeStruct((NC, RA, D), jnp.float32),
        scratch_types=[
            pltpu.VMEM((2, C, EB), jnp.int32),
            pltpu.VMEM((2, C, EB), jnp.int32),
            pltpu.VMEM((EB, D), jnp.float32),
            pltpu.VMEM((EB, D), jnp.float32),
            pltpu.VMEM((EB, D), jnp.float32),
            pltpu.VMEM_SHARED((RA, D), jnp.float32),
            pltpu.SemaphoreType.DMA,
            pltpu.SemaphoreType.DMA,
            pltpu.SemaphoreType.DMA,
            pltpu.SemaphoreType.DMA,
        ],
    )
    def agg_kernel(src_hbm, dst_hbm, h2_hbm, out_hbm,
                   srcc, dstc, bufa, bufb, bufc, agg_sh,
                   sema, semb, semc, semi):
        c = lax.axis_index("c")
        s = lax.axis_index("s")
        w = c * NS + s

        def _zero(i, carry):
            for k in range(D // LANES):
                bufa[i, pl.ds(k * LANES, LANES)] = jnp.zeros((LANES,),
                                                             jnp.float32)
            return carry

        lax.fori_loop(0, EB, _zero, 0)
        base = s * RAS
        for off, ln in _CHUNKS:
            pltpu.sync_copy(bufa.at[pl.ds(0, ln)],
                            agg_sh.at[pl.ds(base + off, ln)])
        pltpu.sync_copy(src_hbm.at[w, pl.ds(0, C)], srcc.at[0])
        pltpu.sync_copy(dst_hbm.at[w, pl.ds(0, C)], dstc.at[0])
        plsc.subcore_barrier()

        def _gref(j):
            return srcc.at[lax.rem(lax.div(j, C), 2), lax.rem(j, C)]

        def _sref(j):
            return dstc.at[lax.rem(lax.div(j, C), 2), lax.rem(j, C)]

        pltpu.async_copy(h2_hbm.at[_gref(0)], bufa, sema)
        pltpu.async_copy(h2_hbm.at[_gref(1)], bufb, semb)
        pltpu.async_copy(h2_hbm.at[_gref(2)], bufc, semc)

        def _triple(t, carry):
            j = t * 3
            g = lax.div(j, C)

            @pl.when(jnp.logical_and(lax.rem(j, C) < 3, g + 1 < nch))
            def _():
                pltpu.async_copy(src_hbm.at[w, pl.ds((g + 1) * C, C)],
                                 srcc.at[lax.rem(g + 1, 2)], semi)
                pltpu.async_copy(dst_hbm.at[w, pl.ds((g + 1) * C, C)],
                                 dstc.at[lax.rem(g + 1, 2)], semi)

            jc = lax.rem(j, C)

            @pl.when(jnp.logical_and(jnp.logical_and(jc >= 3, jc < 6),
                                     g + 1 < nch))
            def _():
                pltpu.make_async_copy(src_hbm.at[w, pl.ds((g + 1) * C, C)],
                                      srcc.at[lax.rem(g + 1, 2)], semi).wait()
                pltpu.make_async_copy(dst_hbm.at[w, pl.ds((g + 1) * C, C)],
                                      dstc.at[lax.rem(g + 1, 2)], semi).wait()

            pltpu.make_async_copy(h2_hbm.at[_gref(j)], bufa, sema).wait()
            pltpu.sync_copy(bufa, agg_sh.at[_sref(j)], add=True)

            @pl.when(j + 3 < nb)
            def _():
                pltpu.async_copy(h2_hbm.at[_gref(j + 3)], bufa, sema)

            pltpu.make_async_copy(h2_hbm.at[_gref(j + 1)], bufb, semb).wait()
            pltpu.sync_copy(bufb, agg_sh.at[_sref(j + 1)], add=True)

            @pl.when(j + 4 < nb)
            def _():
                pltpu.async_copy(h2_hbm.at[_gref(j + 4)], bufb, semb)

            pltpu.make_async_copy(h2_hbm.at[_gref(j + 2)], bufc, semc).wait()
            pltpu.sync_copy(bufc, agg_sh.at[_sref(j + 2)], add=True)

            @pl.when(j + 5 < nb)
            def _():
                pltpu.async_copy(h2_hbm.at[_gref(j + 5)], bufc, semc)

            return carry

        lax.fori_loop(0, nb // 3, _triple, 0)
        plsc.subcore_barrier()
        pltpu.sync_copy(agg_sh.at[pl.ds(base, RAS)],
                        out_hbm.at[c, pl.ds(base, RAS)])

    return agg_kernel


def _mm_body(x_ref, w_ref, o_ref):
    o_ref[...] = jnp.dot(x_ref[...], w_ref[...],
                         preferred_element_type=jnp.float32)


_mm_call = pl.pallas_call(
    _mm_body,
    grid=(N_NODES // ROWB,),
    in_specs=[
        pl.BlockSpec((ROWB, D), lambda i: (i, 0)),
        pl.BlockSpec((D, D), lambda i: (0, 0)),
    ],
    out_specs=pl.BlockSpec((ROWB, D), lambda i: (i, 0)),
    out_shape=jax.ShapeDtypeStruct((N_NODES, D), jnp.float32),
)


def _h2_body(h_ref, degp_ref, o_ref):
    dg = degp_ref[...]
    deg = dg[0, :, 0:1] + dg[1, :, 0:1] + 1.0
    o_ref[...] = h_ref[...] * lax.rsqrt(deg)


_h2_call = pl.pallas_call(
    _h2_body,
    grid=(N_NODES // ROWB,),
    in_specs=[
        pl.BlockSpec((ROWB, D), lambda i: (i, 0)),
        pl.BlockSpec((NC, ROWB, LANES), lambda i: (0, i, 0)),
    ],
    out_specs=pl.BlockSpec((ROWB, D), lambda i: (i, 0)),
    out_shape=jax.ShapeDtypeStruct((N_NODES, D), jnp.float32),
)


def _mlp_body(aggp_ref, degp_ref, h2_ref, bg_ref, w1_ref, b1_ref, w2_ref,
              b2_ref, o_ref):
    p = aggp_ref[...]
    dg = degp_ref[...]
    deg = dg[0, :, 0:1] + dg[1, :, 0:1] + 1.0
    dinv = lax.rsqrt(deg)
    t = (p[0] + p[1] + h2_ref[...]) * dinv
    a = jnp.maximum(t + bg_ref[...], 0.0)
    m = jnp.maximum(
        jnp.dot(a, w1_ref[...], preferred_element_type=jnp.float32)
        + b1_ref[...], 0.0)
    o = jnp.dot(m, w2_ref[...], preferred_element_type=jnp.float32) + b2_ref[...]
    mx = jnp.max(o, axis=1, keepdims=True)
    lse = mx + jnp.log(jnp.sum(jnp.exp(o - mx), axis=1, keepdims=True))
    o_ref[...] = o - lse


_mlp_call = pl.pallas_call(
    _mlp_body,
    grid=(N_NODES // ROWB,),
    in_specs=[
        pl.BlockSpec((NC, ROWB, D), lambda i: (0, i, 0)),
        pl.BlockSpec((NC, ROWB, LANES), lambda i: (0, i, 0)),
        pl.BlockSpec((ROWB, D), lambda i: (i, 0)),
        pl.BlockSpec((1, D), lambda i: (0, 0)),
        pl.BlockSpec((D, D), lambda i: (0, 0)),
        pl.BlockSpec((1, D), lambda i: (0, 0)),
        pl.BlockSpec((D, DO), lambda i: (0, 0)),
        pl.BlockSpec((1, DO), lambda i: (0, 0)),
    ],
    out_specs=pl.BlockSpec((ROWB, DO), lambda i: (i, 0)),
    out_shape=jax.ShapeDtypeStruct((N_NODES, DO), jnp.float32),
)


def kernel(x, edge_index, W_gcn, b_gcn, W1, b1, W2, b2):
    e = edge_index.shape[1]
    nb = -(-e // (NW * EB))
    nb += (-nb) % 3
    nbp = -(-nb // C) * C
    padn = NW * nb * EB - e

    ei = edge_index.astype(jnp.int32)
    pidx = jnp.arange(padn, dtype=jnp.int32)
    pad_src = (pidx * 131) % N_NODES
    pad_dst = N_NODES + pidx % (RA - N_NODES)
    src = jnp.concatenate([ei[0], pad_src]).reshape(NW, nb, EB)
    dst = jnp.concatenate([ei[1], pad_dst]).reshape(NW, nb, EB)
    if nbp > nb:
        src = jnp.pad(src, ((0, 0), (0, nbp - nb), (0, 0)))
        dst = jnp.pad(dst, ((0, 0), (0, nbp - nb), (0, 0)))

    h = _mm_call(x, W_gcn)
    degp = _make_deg_kernel(nb, nbp)(dst)
    h2 = _h2_call(h, degp)
    aggp = _make_agg_kernel(nb, nbp)(src, dst, h2)
    return _mlp_call(aggp, degp, h2, b_gcn.reshape(1, D), W1,
                     b1.reshape(1, D), W2, b2.reshape(1, DO))


# speedup vs baseline: 1.2664x; 1.0068x over previous
import functools

import jax
import jax.numpy as jnp
from jax import lax
from jax.experimental import pallas as pl
from jax.experimental.pallas import tpu as pltpu
from jax.experimental.pallas import tpu_sc as plsc

N_NODES = 10000
D = 128
DO = 64
NC, NS, LANES = 2, 16, 16
NW = NC * NS
EB = 112
C = 8
R = 10240
RA = 10112
RS = R // NS
RAS = RA // NS
ROWB = 400

_mesh = plsc.VectorSubcoreMesh(core_axis_name="c", subcore_axis_name="s")

_CHUNKS = [(q * EB, EB) for q in range(RAS // EB)]
if RAS % EB:
    _CHUNKS.append((RAS - RAS % EB, RAS % EB))
_ZROWS = 64
_DCHUNKS = [(q * _ZROWS, _ZROWS) for q in range(RS // _ZROWS)]


@functools.cache
def _make_deg_kernel(nb, nbp):
    @functools.partial(
        pl.kernel,
        mesh=_mesh,
        out_type=jax.ShapeDtypeStruct((NC, R, LANES), jnp.float32),
        scratch_types=[
            pltpu.VMEM((nbp, EB), jnp.int32),
            pltpu.VMEM((EB, LANES), jnp.float32),
            pltpu.VMEM((_ZROWS, LANES), jnp.float32),
            pltpu.VMEM_SHARED((R, LANES), jnp.float32),
        ],
    )
    def deg_kernel(dst_hbm, out_hbm, idx_v, ones_v, zeros_v, deg_sh):
        c = lax.axis_index("c")
        s = lax.axis_index("s")
        w = c * NS + s

        def _fill(i, carry):
            ones_v[i, :] = jnp.ones((LANES,), jnp.float32)

            @pl.when(i < _ZROWS)
            def _():
                zeros_v[i, :] = jnp.zeros((LANES,), jnp.float32)

            return carry

        lax.fori_loop(0, EB, _fill, 0)

        base = s * RS
        for off, ln in _DCHUNKS:
            pltpu.sync_copy(zeros_v.at[pl.ds(0, ln)],
                            deg_sh.at[pl.ds(base + off, ln)])
        pltpu.sync_copy(dst_hbm.at[w], idx_v)
        plsc.subcore_barrier()

        def _acc(j, carry):
            pltpu.sync_copy(ones_v, deg_sh.at[idx_v.at[j]], add=True)
            return carry

        lax.fori_loop(0, nb, _acc, 0)
        plsc.subcore_barrier()
        pltpu.sync_copy(deg_sh.at[pl.ds(base, RS)],
                        out_hbm.at[c, pl.ds(base, RS)])

    return deg_kernel


@functools.cache
def _make_agg_kernel(nb, nbp):
    nch = nbp // C

    @functools.partial(
        pl.kernel,
        mesh=_mesh,
        out_type=jax.ShapeDtypeStruct((NC, RA, D), jnp.float32),
        scratch_types=[
            pltpu.VMEM((2, C, EB), jnp.int32),
            pltpu.VMEM((2, C, EB), jnp.int32),
            pltpu.VMEM((EB, D), jnp.float32),
            pltpu.VMEM((EB, D), jnp.float32),
            pltpu.VMEM((EB, D), jnp.float32),
            pltpu.VMEM_SHARED((RA, D), jnp.float32),
            pltpu.SemaphoreType.DMA,
            pltpu.SemaphoreType.DMA,
            pltpu.SemaphoreType.DMA,
            pltpu.SemaphoreType.DMA,
        ],
    )
    def agg_kernel(src_hbm, dst_hbm, h2_hbm, out_hbm,
                   srcc, dstc, bufa, bufb, bufc, agg_sh,
                   sema, semb, semc, semi):
        c = lax.axis_index("c")
        s = lax.axis_index("s")
        w = c * NS + s

        def _zero(i, carry):
            for k in range(D // LANES):
                bufa[i, pl.ds(k * LANES, LANES)] = jnp.zeros((LANES,),
                                                             jnp.float32)
            return carry

        lax.fori_loop(0, EB, _zero, 0)
        base = s * RAS
        for off, ln in _CHUNKS:
            pltpu.sync_copy(bufa.at[pl.ds(0, ln)],
                            agg_sh.at[pl.ds(base + off, ln)])
        pltpu.sync_copy(src_hbm.at[w, pl.ds(0, C)], srcc.at[0])
        pltpu.sync_copy(dst_hbm.at[w, pl.ds(0, C)], dstc.at[0])
        plsc.subcore_barrier()

        def _gref(j):
            return srcc.at[lax.rem(lax.div(j, C), 2), lax.rem(j, C)]

        def _sref(j):
            return dstc.at[lax.rem(lax.div(j, C), 2), lax.rem(j, C)]

        pltpu.async_copy(h2_hbm.at[_gref(0)], bufa, sema)
        pltpu.async_copy(h2_hbm.at[_gref(1)], bufb, semb)
        pltpu.async_copy(h2_hbm.at[_gref(2)], bufc, semc)

        def _triple(t, carry):
            j = t * 3
            g = lax.div(j, C)

            @pl.when(jnp.logical_and(lax.rem(j, C) < 3, g + 1 < nch))
            def _():
                pltpu.async_copy(src_hbm.at[w, pl.ds((g + 1) * C, C)],
                                 srcc.at[lax.rem(g + 1, 2)], semi)
                pltpu.async_copy(dst_hbm.at[w, pl.ds((g + 1) * C, C)],
                                 dstc.at[lax.rem(g + 1, 2)], semi)

            jc = lax.rem(j, C)

            @pl.when(jnp.logical_and(jnp.logical_and(jc >= 3, jc < 6),
                                     g + 1 < nch))
            def _():
                pltpu.make_async_copy(src_hbm.at[w, pl.ds((g + 1) * C, C)],
                                      srcc.at[lax.rem(g + 1, 2)], semi).wait()
                pltpu.make_async_copy(dst_hbm.at[w, pl.ds((g + 1) * C, C)],
                                      dstc.at[lax.rem(g + 1, 2)], semi).wait()

            pltpu.make_async_copy(h2_hbm.at[_gref(j)], bufa, sema).wait()
            pltpu.sync_copy(bufa, agg_sh.at[_sref(j)], add=True)

            @pl.when(j + 3 < nb)
            def _():
                pltpu.async_copy(h2_hbm.at[_gref(j + 3)], bufa, sema)

            pltpu.make_async_copy(h2_hbm.at[_gref(j + 1)], bufb, semb).wait()
            pltpu.sync_copy(bufb, agg_sh.at[_sref(j + 1)], add=True)

            @pl.when(j + 4 < nb)
            def _():
                pltpu.async_copy(h2_hbm.at[_gref(j + 4)], bufb, semb)

            pltpu.make_async_copy(h2_hbm.at[_gref(j + 2)], bufc, semc).wait()
            pltpu.sync_copy(bufc, agg_sh.at[_sref(j + 2)], add=True)

            @pl.when(j + 5 < nb)
            def _():
                pltpu.async_copy(h2_hbm.at[_gref(j + 5)], bufc, semc)

            return carry

        lax.fori_loop(0, nb // 3, _triple, 0)
        plsc.subcore_barrier()
        pltpu.sync_copy(agg_sh.at[pl.ds(base, RAS)],
                        out_hbm.at[c, pl.ds(base, RAS)])

    return agg_kernel


def _h2_body(x_ref, w_ref, degp_ref, o_ref):
    dg = degp_ref[...]
    deg = dg[0, :, 0:1] + dg[1, :, 0:1] + 1.0
    h = jnp.dot(x_ref[...], w_ref[...], preferred_element_type=jnp.float32)
    o_ref[...] = h * lax.rsqrt(deg)


_h2_call = pl.pallas_call(
    _h2_body,
    grid=(N_NODES // ROWB,),
    in_specs=[
        pl.BlockSpec((ROWB, D), lambda i: (i, 0)),
        pl.BlockSpec((D, D), lambda i: (0, 0)),
        pl.BlockSpec((NC, ROWB, LANES), lambda i: (0, i, 0)),
    ],
    out_specs=pl.BlockSpec((ROWB, D), lambda i: (i, 0)),
    out_shape=jax.ShapeDtypeStruct((N_NODES, D), jnp.float32),
)


def _mlp_body(aggp_ref, degp_ref, h2_ref, bg_ref, w1_ref, b1_ref, w2_ref,
              b2_ref, o_ref):
    p = aggp_ref[...]
    dg = degp_ref[...]
    deg = dg[0, :, 0:1] + dg[1, :, 0:1] + 1.0
    dinv = lax.rsqrt(deg)
    t = (p[0] + p[1] + h2_ref[...]) * dinv
    a = jnp.maximum(t + bg_ref[...], 0.0)
    m = jnp.maximum(
        jnp.dot(a, w1_ref[...], preferred_element_type=jnp.float32)
        + b1_ref[...], 0.0)
    o = jnp.dot(m, w2_ref[...], preferred_element_type=jnp.float32) + b2_ref[...]
    mx = jnp.max(o, axis=1, keepdims=True)
    lse = mx + jnp.log(jnp.sum(jnp.exp(o - mx), axis=1, keepdims=True))
    o_ref[...] = o - lse


_mlp_call = pl.pallas_call(
    _mlp_body,
    grid=(N_NODES // ROWB,),
    in_specs=[
        pl.BlockSpec((NC, ROWB, D), lambda i: (0, i, 0)),
        pl.BlockSpec((NC, ROWB, LANES), lambda i: (0, i, 0)),
        pl.BlockSpec((ROWB, D), lambda i: (i, 0)),
        pl.BlockSpec((1, D), lambda i: (0, 0)),
        pl.BlockSpec((D, D), lambda i: (0, 0)),
        pl.BlockSpec((1, D), lambda i: (0, 0)),
        pl.BlockSpec((D, DO), lambda i: (0, 0)),
        pl.BlockSpec((1, DO), lambda i: (0, 0)),
    ],
    out_specs=pl.BlockSpec((ROWB, DO), lambda i: (i, 0)),
    out_shape=jax.ShapeDtypeStruct((N_NODES, DO), jnp.float32),
)


def kernel(x, edge_index, W_gcn, b_gcn, W1, b1, W2, b2):
    e = edge_index.shape[1]
    nb = -(-e // (NW * EB))
    nb += (-nb) % 3
    nbp = -(-nb // C) * C
    padn = NW * nb * EB - e

    ei = edge_index.astype(jnp.int32)
    pidx = jnp.arange(padn, dtype=jnp.int32)
    pad_src = (pidx * 131) % N_NODES
    pad_dst = N_NODES + pidx % (RA - N_NODES)
    sd = jnp.concatenate([ei, jnp.stack([pad_src, pad_dst])], axis=1)
    sd = sd.reshape(2, NW, nb, EB)
    if nbp > nb:
        sd = jnp.pad(sd, ((0, 0), (0, 0), (0, nbp - nb), (0, 0)))
    src, dst = sd[0], sd[1]

    degp = _make_deg_kernel(nb, nbp)(dst)
    h2 = _h2_call(x, W_gcn, degp)
    aggp = _make_agg_kernel(nb, nbp)(src, dst, h2)
    return _mlp_call(aggp, degp, h2, b_gcn.reshape(1, D), W1,
                     b1.reshape(1, D), W2, b2.reshape(1, DO))
